# baseline (speedup 1.0000x reference)
import functools

import jax
import jax.numpy as jnp
from jax import lax
from jax.experimental import pallas as pl
from jax.experimental.pallas import tpu as pltpu
from jax.experimental.pallas import tpu_sc as plsc

SEQ, BATCH, DIM = 8192, 4, 2048
NC, NS = 2, 16
NW = NC * NS
ROWS_PER_W = SEQ // NW
R = 4
CHUNKS = ROWS_PER_W // R


def _sc_body(x_hbm, w_hbm, out_hbm, ybuf, wbuf):
    cid = lax.axis_index("c")
    sid = lax.axis_index("s")
    base = (cid * NS + sid) * ROWS_PER_W

    def chunk(i, carry):
        row0 = base + i * R
        pltpu.sync_copy(x_hbm.at[pl.ds(row0, R)], ybuf)
        pltpu.sync_copy(w_hbm.at[pl.ds(row0, R)], wbuf)
        for r in range(R):
            def jbody(j, c):
                off = j * 16
                wv = wbuf[r, pl.ds(off, 16)]
                for b in range(BATCH):
                    ybuf[r, b, pl.ds(off, 16)] = ybuf[r, b, pl.ds(off, 16)] + wv
                return c
            lax.fori_loop(0, DIM // 16, jbody, 0)
        pltpu.sync_copy(ybuf, out_hbm.at[pl.ds(row0, R)])
        return carry

    lax.fori_loop(0, CHUNKS, chunk, 0)


@functools.partial(
    pl.kernel,
    mesh=plsc.VectorSubcoreMesh(core_axis_name="c", subcore_axis_name="s"),
    out_type=jax.ShapeDtypeStruct((SEQ, BATCH, DIM), jnp.float32),
    scratch_types=[
        pltpu.VMEM((R, BATCH, DIM), jnp.float32),
        pltpu.VMEM((R, DIM), jnp.float32),
    ],
)
def _sc_add(x_hbm, w_hbm, out_hbm, ybuf, wbuf):
    _sc_body(x_hbm, w_hbm, out_hbm, ybuf, wbuf)


def kernel(x, weight):
    return _sc_add(x, weight[:SEQ])
